# baseline (speedup 1.0000x reference)
import functools

import jax
import jax.numpy as jnp
from jax import lax
from jax.experimental import pallas as pl
from jax.experimental.pallas import tpu as pltpu
from jax.experimental.pallas import tpu_sc as plsc


_CHUNK_ROWS = 4


def kernel(embeddings, pos_table):
    Bn, Tn, Hn = embeddings.shape
    info = plsc.get_sparse_core_info()
    nw = info.num_cores * info.num_subcores
    rows_w = Tn // nw
    chunks = rows_w // _CHUNK_ROWS
    nsl = _CHUNK_ROWS * Hn // 16
    mesh = plsc.VectorSubcoreMesh(core_axis_name="c", subcore_axis_name="s")

    @functools.partial(
        pl.kernel,
        mesh=mesh,
        out_type=jax.ShapeDtypeStruct((Bn, Tn, Hn), jnp.float32),
        scratch_types=[
            pltpu.VMEM((2, _CHUNK_ROWS, Hn), jnp.float32),
            pltpu.VMEM((2, Bn, _CHUNK_ROWS, Hn), jnp.float32),
            pltpu.SemaphoreType.DMA((2,)),
            pltpu.SemaphoreType.DMA((2,)),
        ],
    )
    def sc_k(emb_hbm, pos_hbm, out_hbm, pos_v, emb_v, lsem, ssem):
        wid = lax.axis_index("s") * info.num_cores + lax.axis_index("c")
        t0 = wid * rows_w

        def issue_loads(c, p):
            row = t0 + c * _CHUNK_ROWS
            cps = [pltpu.async_copy(
                pos_hbm.at[pl.ds(row, _CHUNK_ROWS)], pos_v.at[p], lsem.at[p])]
            for b in range(Bn):
                cps.append(pltpu.async_copy(
                    emb_hbm.at[b, pl.ds(row, _CHUNK_ROWS)], emb_v.at[p, b],
                    lsem.at[p]))
            return cps

        def issue_stores(c, p):
            row = t0 + c * _CHUNK_ROWS
            return [pltpu.async_copy(
                emb_v.at[p, b], out_hbm.at[b, pl.ds(row, _CHUNK_ROWS)],
                ssem.at[p]) for b in range(Bn)]

        loads = {0: issue_loads(0, 0)}
        stores = {}
        for c in range(chunks):
            p = c % 2
            for cp in loads.pop(c):
                cp.wait()
            if c + 1 < chunks:
                if c - 1 in stores:
                    for cp in stores.pop(c - 1):
                        cp.wait()
                loads[c + 1] = issue_loads(c + 1, (c + 1) % 2)

            def body(j, _):
                r = j // (Hn // 16)
                col = (j % (Hn // 16)) * 16
                pv = pos_v[p, r, pl.ds(col, 16)]
                for b in range(Bn):
                    emb_v[p, b, r, pl.ds(col, 16)] = (
                        emb_v[p, b, r, pl.ds(col, 16)] + pv)
                return 0

            lax.fori_loop(0, nsl, body, 0)
            stores[c] = issue_stores(c, p)
        for cps in stores.values():
            for cp in cps:
                cp.wait()

    return sc_k(embeddings, pos_table)
